# Initial kernel scaffold; baseline (speedup 1.0000x reference)
import jax
import jax.numpy as jnp
from jax.experimental import pallas as pl


def kernel(stacked_radii, cell, gt_rdf):
    raise NotImplementedError("write your pallas kernel here")


# speedup vs baseline: 1880.0274x; 1880.0274x over previous
import functools

import jax
import jax.numpy as jnp
import numpy as np
from jax import lax
from jax.experimental import pallas as pl
from jax.experimental.pallas import tpu as pltpu
from jax.experimental.pallas import tpu_sc as plsc

XLIM = 10.0
DR = 0.1
N_BINS = int(XLIM / DR)
BINS = np.linspace(1e-06, XLIM, N_BINS + 1)

N_REPLICAS = 8
N_ATOMS = 128
N_FRAMES = 256

NW = 32
W_PER_R = NW // N_REPLICAS
F_PER_W = N_FRAMES // W_PER_R
NCHUNK = N_ATOMS // 16

TAB_N = 16384
SQ_MAX = 75.5
HIST_PAD = 112


def _build_tables():
    e32 = BINS.astype(np.float32)
    u = np.zeros(HIST_PAD, np.float32)
    for k in range(N_BINS + 1):
        x = np.float32(np.float64(e32[k]) ** 2)
        while np.sqrt(x) >= e32[k]:
            x = np.nextafter(x, np.float32(0.0), dtype=np.float32)
        while np.sqrt(x) < e32[k]:
            x = np.nextafter(x, np.float32(np.inf), dtype=np.float32)
        u[k] = x
    u[N_BINS + 1:] = np.float32(np.inf)
    tsc = np.float32(TAB_N / SQ_MAX)
    xm = np.arange(TAB_N, dtype=np.float64) / np.float64(tsc) - 1e-4
    tab = np.searchsorted(u[1:N_BINS + 1].astype(np.float64), xm,
                          side='right').astype(np.int32)
    assert np.all(np.diff(tab) <= 1)
    return tab, u, tsc


_TAB_NP, _U_NP, _TSC = _build_tables()


def _rne_bf16(v):
    u = plsc.bitcast(v, jnp.int32)
    r = u + jnp.int32(0x7FFF) + (lax.shift_right_logical(u, 16) & 1)
    return plsc.bitcast(r & jnp.int32(-65536), jnp.float32)


def _sc_body(pos_hbm, aux_hbm, tab_hbm, u_hbm, out_hbm,
             pos_v, aux_v, tab_v, u_v, posd_v, histl_v, hist_v):
    wid = lax.axis_index("s") * 2 + lax.axis_index("c")
    rep = wid // W_PER_R
    f0 = (wid % W_PER_R) * F_PER_W

    pltpu.sync_copy(tab_hbm, tab_v)
    pltpu.sync_copy(u_hbm, u_v)
    pltpu.sync_copy(aux_hbm, aux_v)

    zeros = jnp.zeros((16,), jnp.float32)

    def zero_body(k, _):
        histl_v[pl.ds(k * 16, 16)] = zeros
        return 0

    lax.fori_loop(0, NCHUNK * HIST_PAD, zero_body, 0)

    lane = lax.iota(jnp.int32, 16)
    tsc_b = jnp.full((16,), _TSC, jnp.float32)
    u0_b = jnp.full((16,), _U_NP[0], jnp.float32)
    mcap = jnp.full((16,), TAB_N - 1, jnp.int32)
    lxb = aux_v[0, pl.ds(0, 16)]
    lyb = aux_v[1, pl.ds(0, 16)]
    lzb = aux_v[2, pl.ds(0, 16)]
    one_b = jnp.ones((16,), jnp.float32)
    laneoff = [lane + c * (HIST_PAD * 16) for c in range(NCHUNK)]

    def frame_body(t, _):
        pltpu.sync_copy(pos_hbm.at[rep, f0 + t], pos_v)
        fx, fy, fz = [], [], []
        for comp, dst in ((0, fx), (1, fy), (2, fz)):
            for k in range(NCHUNK):
                v = pos_v[comp, pl.ds(k * 16, 16)]
                dst.append(v)
                posd_v[pl.ds(comp * 256 + k * 16, 16)] = v
                posd_v[pl.ds(comp * 256 + 128 + k * 16, 16)] = v

        def pair_pass(o, wb):
            for c in range(NCHUNK):
                sx = posd_v[pl.ds(c * 16 + o, 16)]
                sy = posd_v[pl.ds(256 + c * 16 + o, 16)]
                sz = posd_v[pl.ds(512 + c * 16 + o, 16)]
                ax = jnp.abs(fx[c] - sx)
                ay = jnp.abs(fy[c] - sy)
                az = jnp.abs(fz[c] - sz)
                tx = _rne_bf16(jnp.minimum(ax, one_b - ax)) * lxb
                ty = _rne_bf16(jnp.minimum(ay, one_b - ay)) * lyb
                tz = _rne_bf16(jnp.minimum(az, one_b - az)) * lzb
                sq = (tx * tx + ty * ty) + tz * tz
                m = jnp.minimum((sq * tsc_b).astype(jnp.int32), mcap)
                t0 = plsc.load_gather(tab_v, [m])
                t1 = t0 + 1
                up = plsc.load_gather(u_v, [t1])
                idx = jnp.where(sq >= up, t1, t0)
                addr = idx * 16 + laneoff[c]
                plsc.addupdate_scatter(histl_v, [addr], wb,
                                       mask=sq >= u0_b)

        ones = jnp.ones((16,), jnp.float32)
        halves = jnp.full((16,), 0.5, jnp.float32)

        def o_body(o, _):
            pair_pass(o, ones)
            return 0

        lax.fori_loop(1, 64, o_body, 0)
        pair_pass(64, halves)
        return 0

    lax.fori_loop(0, F_PER_W, frame_body, 0)

    base = lane * 16

    def red_body(cb, _):
        acc = jnp.zeros((16,), jnp.float32)
        for c in range(NCHUNK):
            for l in range(16):
                acc = acc + plsc.load_gather(
                    histl_v, [base + (c * (HIST_PAD * 16) + cb * 256 + l)])
        hist_v[pl.ds(cb * 16, 16)] = acc + acc
        return 0

    lax.fori_loop(0, HIST_PAD // 16, red_body, 0)
    pltpu.sync_copy(hist_v, out_hbm.at[wid])


_sc_hist = functools.partial(
    pl.kernel,
    out_type=jax.ShapeDtypeStruct((NW, HIST_PAD), jnp.float32),
    mesh=plsc.VectorSubcoreMesh(core_axis_name="c", subcore_axis_name="s",
                                num_cores=2, num_subcores=16),
    scratch_types=[
        pltpu.VMEM((3, N_ATOMS), jnp.float32),
        pltpu.VMEM((3, 16), jnp.float32),
        pltpu.VMEM((TAB_N,), jnp.int32),
        pltpu.VMEM((HIST_PAD,), jnp.float32),
        pltpu.VMEM((3 * 2 * N_ATOMS,), jnp.float32),
        pltpu.VMEM((NCHUNK * HIST_PAD * 16,), jnp.float32),
        pltpu.VMEM((HIST_PAD,), jnp.float32),
    ],
    compiler_params=pltpu.CompilerParams(needs_layout_passes=False),
)(_sc_body)


def kernel(stacked_radii, cell, gt_rdf):
    inv = jnp.linalg.inv(cell)
    frac = stacked_radii @ inv
    pos = jnp.transpose(frac, (1, 0, 3, 2)).astype(jnp.float32)
    diag = jnp.diagonal(cell).astype(jnp.bfloat16).astype(jnp.float32)
    aux = jnp.tile(diag[:, None], (1, 16))
    tab = jnp.asarray(_TAB_NP)
    u = jnp.asarray(_U_NP)
    parts = _sc_hist(pos, aux, tab, u)
    hist = parts.reshape(N_REPLICAS, W_PER_R, HIST_PAD).sum(axis=1)[:, :N_BINS]

    bins_arr = jnp.asarray(BINS, dtype=jnp.float32)
    volume = jnp.abs(jnp.linalg.det(cell))
    rho = (N_FRAMES * N_ATOMS * N_ATOMS) / volume
    shell = rho * (4.0 / 3.0) * jnp.pi * (bins_arr[1:] ** 3 - bins_arr[:-1] ** 3)
    rdfs = hist / shell
    maes = XLIM * jnp.mean(jnp.abs(rdfs - gt_rdf), axis=1)
    return (rdfs, maes)
